# baseline (speedup 1.0000x reference)
import functools

import jax
import jax.numpy as jnp
from jax import lax
from jax.experimental import pallas as pl
from jax.experimental.pallas import tpu as pltpu
from jax.experimental.pallas import tpu_sc as plsc

V = 12224
D = 11 * 1 * 256
B = 4096
NC, NS = 2, 16
NW = NC * NS
BPW = B // NW
C = 16
NCH = BPW // C

_mesh = plsc.VectorSubcoreMesh(core_axis_name="c", subcore_axis_name="s")


@functools.partial(
    pl.kernel,
    mesh=_mesh,
    out_type=jax.ShapeDtypeStruct((B, D), jnp.float32),
    scratch_types=[
        pltpu.VMEM((NCH, C), jnp.int32),
        pltpu.VMEM((C, D), jnp.float32),
        pltpu.VMEM((C, D), jnp.float32),
        pltpu.SemaphoreType.DMA,
        pltpu.SemaphoreType.DMA,
        pltpu.SemaphoreType.DMA,
        pltpu.SemaphoreType.DMA,
    ],
)
def _sc_gather(table_hbm, idx_hbm, out_hbm, idx_v, buf0, buf1, g0, g1, o0, o1):
    wid = lax.axis_index("s") * NC + lax.axis_index("c")
    base = wid * BPW
    pltpu.sync_copy(idx_hbm.at[wid], idx_v)

    bufs = (buf0, buf1)
    gsems = (g0, g1)
    osems = (o0, o1)
    gops = [None] * NCH
    oops = [None] * NCH

    gops[0] = pltpu.async_copy(table_hbm.at[idx_v.at[0]], bufs[0], gsems[0])
    for ci in range(NCH):
        p = ci & 1
        if ci + 1 < NCH:
            if ci >= 1:
                oops[ci - 1].wait()
            gops[ci + 1] = pltpu.async_copy(
                table_hbm.at[idx_v.at[ci + 1]], bufs[1 - p], gsems[1 - p]
            )
        gops[ci].wait()
        oops[ci] = pltpu.async_copy(
            bufs[p], out_hbm.at[pl.ds(base + ci * C, C)], osems[p]
        )
    oops[NCH - 2].wait()
    oops[NCH - 1].wait()


def kernel(table, indices):
    tbl = table.reshape(V, D)
    idx = indices.astype(jnp.int32).reshape(NW, NCH, C)
    out = _sc_gather(tbl, idx)
    return out.reshape(B, 11, 1, 256)


# speedup vs baseline: 20.2693x; 20.2693x over previous
import functools

import jax
import jax.numpy as jnp
from jax import lax
from jax.experimental import pallas as pl
from jax.experimental.pallas import tpu as pltpu
from jax.experimental.pallas import tpu_sc as plsc

V = 12224
D = 11 * 1 * 256
B = 4096
NC, NS = 2, 16
NW = NC * NS
BPW = B // NW
C = 16
NCH = BPW // C

_mesh = plsc.VectorSubcoreMesh(core_axis_name="c", subcore_axis_name="s")


@functools.partial(
    pl.kernel,
    mesh=_mesh,
    out_type=jax.ShapeDtypeStruct((B, 11, 1, 256), jnp.float32),
    scratch_types=[
        pltpu.VMEM((NCH, C), jnp.int32),
        pltpu.VMEM((C, 11, 1, 256), jnp.float32),
        pltpu.VMEM((C, 11, 1, 256), jnp.float32),
        pltpu.SemaphoreType.DMA,
        pltpu.SemaphoreType.DMA,
        pltpu.SemaphoreType.DMA,
        pltpu.SemaphoreType.DMA,
    ],
)
def _sc_gather(table_hbm, idx_hbm, out_hbm, idx_v, buf0, buf1, g0, g1, o0, o1):
    wid = lax.axis_index("s") * NC + lax.axis_index("c")
    base = wid * BPW
    pltpu.sync_copy(idx_hbm.at[wid], idx_v)

    bufs = (buf0, buf1)
    gsems = (g0, g1)
    osems = (o0, o1)
    gops = [None] * NCH
    oops = [None] * NCH

    gops[0] = pltpu.async_copy(table_hbm.at[idx_v.at[0]], bufs[0], gsems[0])
    for ci in range(NCH):
        p = ci & 1
        if ci + 1 < NCH:
            if ci >= 1:
                oops[ci - 1].wait()
            gops[ci + 1] = pltpu.async_copy(
                table_hbm.at[idx_v.at[ci + 1]], bufs[1 - p], gsems[1 - p]
            )
        gops[ci].wait()
        oops[ci] = pltpu.async_copy(
            bufs[p], out_hbm.at[pl.ds(base + ci * C, C)], osems[p]
        )
    oops[NCH - 2].wait()
    oops[NCH - 1].wait()


def kernel(table, indices):
    idx = indices.astype(jnp.int32).reshape(NW, NCH, C)
    return _sc_gather(table, idx)


# speedup vs baseline: 20.3926x; 1.0061x over previous
import functools

import jax
import jax.numpy as jnp
from jax import lax
from jax.experimental import pallas as pl
from jax.experimental.pallas import tpu as pltpu
from jax.experimental.pallas import tpu_sc as plsc

V = 12224
ROW = (11, 1, 256)
B = 4096
NC, NS = 2, 16
NW = NC * NS
BPW = B // NW
C = 8
NCH = BPW // C
NBUF = 4

_mesh = plsc.VectorSubcoreMesh(core_axis_name="c", subcore_axis_name="s")


@functools.partial(
    pl.kernel,
    mesh=_mesh,
    out_type=jax.ShapeDtypeStruct((B,) + ROW, jnp.float32),
    scratch_types=[
        pltpu.VMEM((NCH, C), jnp.int32),
    ]
    + [pltpu.VMEM((C,) + ROW, jnp.float32) for _ in range(NBUF)]
    + [pltpu.SemaphoreType.DMA for _ in range(2 * NBUF)],
)
def _sc_gather(table_hbm, idx_hbm, out_hbm, idx_v, *bufs_and_sems):
    bufs = bufs_and_sems[:NBUF]
    gsems = bufs_and_sems[NBUF : 2 * NBUF]
    osems = bufs_and_sems[2 * NBUF :]

    wid = lax.axis_index("s") * NC + lax.axis_index("c")
    base = wid * BPW
    pltpu.sync_copy(idx_hbm.at[wid], idx_v)

    gops = [None] * NCH
    oops = [None] * NCH
    for ci in range(NBUF):
        gops[ci] = pltpu.async_copy(
            table_hbm.at[idx_v.at[ci]], bufs[ci], gsems[ci]
        )
    for ci in range(NCH):
        p = ci % NBUF
        gops[ci].wait()
        oops[ci] = pltpu.async_copy(
            bufs[p], out_hbm.at[pl.ds(base + ci * C, C)], osems[p]
        )
        nxt = ci + NBUF
        if nxt < NCH:
            oops[ci].wait()
            gops[nxt] = pltpu.async_copy(
                table_hbm.at[idx_v.at[nxt]], bufs[p], gsems[p]
            )
    for ci in range(NCH - NBUF, NCH):
        oops[ci].wait()


def kernel(table, indices):
    idx = indices.astype(jnp.int32).reshape(NW, NCH, C)
    return _sc_gather(table, idx)


# speedup vs baseline: 20.6940x; 1.0148x over previous
import functools

import jax
import jax.numpy as jnp
from jax import lax
from jax.experimental import pallas as pl
from jax.experimental.pallas import tpu as pltpu
from jax.experimental.pallas import tpu_sc as plsc

V = 12224
ROW = (11, 1, 256)
B = 4096
NC, NS = 2, 16
NW = NC * NS
BPW = B // NW
C = 8
NCH = BPW // C
NBUF = 5

_mesh = plsc.VectorSubcoreMesh(core_axis_name="c", subcore_axis_name="s")


@functools.partial(
    pl.kernel,
    mesh=_mesh,
    out_type=jax.ShapeDtypeStruct((B,) + ROW, jnp.float32),
    scratch_types=[
        pltpu.VMEM((NCH, C), jnp.int32),
    ]
    + [pltpu.VMEM((C,) + ROW, jnp.float32) for _ in range(NBUF)]
    + [pltpu.SemaphoreType.DMA for _ in range(2 * NBUF)],
)
def _sc_gather(table_hbm, idx_hbm, out_hbm, idx_v, *bufs_and_sems):
    bufs = bufs_and_sems[:NBUF]
    gsems = bufs_and_sems[NBUF : 2 * NBUF]
    osems = bufs_and_sems[2 * NBUF :]

    wid = lax.axis_index("s") * NC + lax.axis_index("c")
    base = wid * BPW
    pltpu.sync_copy(idx_hbm.at[wid], idx_v)

    gops = [None] * NCH
    oops = [None] * NCH
    for ci in range(NBUF):
        gops[ci] = pltpu.async_copy(
            table_hbm.at[idx_v.at[ci]], bufs[ci], gsems[ci]
        )
    for ci in range(NCH):
        p = ci % NBUF
        gops[ci].wait()
        oops[ci] = pltpu.async_copy(
            bufs[p], out_hbm.at[pl.ds(base + ci * C, C)], osems[p]
        )
        nxt = ci + NBUF
        if nxt < NCH:
            oops[ci].wait()
            gops[nxt] = pltpu.async_copy(
                table_hbm.at[idx_v.at[nxt]], bufs[p], gsems[p]
            )
    for ci in range(NCH - NBUF, NCH):
        oops[ci].wait()


def kernel(table, indices):
    idx = indices.astype(jnp.int32).reshape(NW, NCH, C)
    return _sc_gather(table, idx)


# speedup vs baseline: 20.8353x; 1.0068x over previous
import functools

import jax
import jax.numpy as jnp
from jax import lax
from jax.experimental import pallas as pl
from jax.experimental.pallas import tpu as pltpu
from jax.experimental.pallas import tpu_sc as plsc

V = 12224
ROW = (11, 1, 256)
B = 4096
NC, NS = 2, 16
NW = NC * NS
BPW = B // NW
C = 8
NCH = BPW // C
NBUF = 5

_mesh = plsc.VectorSubcoreMesh(core_axis_name="c", subcore_axis_name="s")


@functools.partial(
    pl.kernel,
    mesh=_mesh,
    out_type=jax.ShapeDtypeStruct((B,) + ROW, jnp.float32),
    scratch_types=[
        pltpu.VMEM((BPW,), jnp.int32),
    ]
    + [pltpu.VMEM((C,) + ROW, jnp.float32) for _ in range(NBUF)]
    + [pltpu.SemaphoreType.DMA for _ in range(2 * NBUF)],
)
def _sc_gather(table_hbm, idx_hbm, out_hbm, idx_v, *bufs_and_sems):
    bufs = bufs_and_sems[:NBUF]
    gsems = bufs_and_sems[NBUF : 2 * NBUF]
    osems = bufs_and_sems[2 * NBUF :]

    wid = lax.axis_index("s") * NC + lax.axis_index("c")
    base = wid * BPW
    pltpu.sync_copy(idx_hbm.at[pl.ds(base, BPW)], idx_v)

    gops = [None] * NCH
    oops = [None] * NCH
    for ci in range(NBUF):
        gops[ci] = pltpu.async_copy(
            table_hbm.at[idx_v.at[pl.ds(ci * C, C)]], bufs[ci], gsems[ci]
        )
    for ci in range(NCH):
        p = ci % NBUF
        gops[ci].wait()
        oops[ci] = pltpu.async_copy(
            bufs[p], out_hbm.at[pl.ds(base + ci * C, C)], osems[p]
        )
        nxt = ci + NBUF
        if nxt < NCH:
            oops[ci].wait()
            gops[nxt] = pltpu.async_copy(
                table_hbm.at[idx_v.at[pl.ds(nxt * C, C)]], bufs[p], gsems[p]
            )
    for ci in range(NCH - NBUF, NCH):
        oops[ci].wait()


def kernel(table, indices):
    return _sc_gather(table, indices.astype(jnp.int32))
